# baseline (speedup 1.0000x reference)
import functools

import jax
import jax.numpy as jnp
from jax import lax
from jax.experimental import pallas as pl
from jax.experimental.pallas import tpu as pltpu
import jax.experimental.pallas.tpu_sc as plsc

_ROT0 = (13, 15, 26, 6)
_ROT1 = (17, 29, 16, 24)
_TINY = float.fromhex("0x1p-126")
_NEG_BIG = -3.0e38
_SUB = 256
_CHUNK = 32768
_BUF = 16384
_SC_TAIL = 279104


def _log2(x):
    return jnp.log2(x)


def _rotl(x, d):
    return lax.shift_left(x, jnp.uint32(d)) | lax.shift_right_logical(
        x, jnp.uint32(32 - d)
    )


def _threefry_bits(i):
    ks = (0, 42, 42 ^ 0x1BD11BDA)
    x1 = i + jnp.uint32(42)
    x0 = x1
    x1 = _rotl(x1, _ROT0[0]) ^ x0
    for r in _ROT0[1:]:
        x0 = x0 + x1
        x1 = _rotl(x1, r)
        x1 = x1 ^ x0
    x0 = x0 + jnp.uint32(ks[1] & 0xFFFFFFFF)
    x1 = x1 + jnp.uint32((ks[2] + 1) & 0xFFFFFFFF)
    for g in range(1, 5):
        rots = _ROT0 if g % 2 == 0 else _ROT1
        for r in rots:
            x0 = x0 + x1
            x1 = _rotl(x1, r)
            x1 = x1 ^ x0
        x0 = x0 + jnp.uint32(ks[(g + 1) % 3] & 0xFFFFFFFF)
        x1 = x1 + jnp.uint32((ks[(g + 2) % 3] + g + 1) & 0xFFFFFFFF)
    return x0 ^ x1


def _gumbel_from_bits(bits):
    fb = lax.shift_right_logical(bits, jnp.uint32(9)) | jnp.uint32(0x3F800000)
    u = lax.bitcast_convert_type(fb, jnp.float32) - jnp.float32(1.0)
    uu = u + _TINY
    nln2 = jnp.float32(-0.6931471805599453)
    m1 = _log2(uu) * nln2
    return _log2(m1) * nln2


def _sc_bits_body(o_ref, buf_ref, sem, *, ncols, c0, ltail):
    c = lax.axis_index("c")
    s = lax.axis_index("s")
    r = c * 16 + s
    base = r * ncols + c0
    iota16 = lax.iota(jnp.int32, 16)
    nfull = ltail // _BUF
    rem = ltail - nfull * _BUF

    def emit(off, nvec):
        @plsc.parallel_loop(0, nvec, unroll=4)
        def _gen(v):
            ctr = (base + off + v * 16 + iota16).astype(jnp.uint32)
            buf_ref[pl.ds(v * 16, 16)] = _threefry_bits(ctr)

        cp = pltpu.make_async_copy(
            buf_ref.at[pl.ds(0, nvec * 16)],
            o_ref.at[r, pl.ds(off, nvec * 16)],
            sem,
        )
        cp.start()
        cp.wait()

    def chunk(k, carry):
        emit(k * _BUF, _BUF // 16)
        return carry

    lax.fori_loop(0, nfull, chunk, 0)
    if rem:
        emit(nfull * _BUF, rem // 16)


def _sc_bits(nrows, ncols, c0, ltail):
    mesh = plsc.VectorSubcoreMesh(core_axis_name="c", subcore_axis_name="s")
    f = pl.kernel(
        functools.partial(_sc_bits_body, ncols=ncols, c0=c0, ltail=ltail),
        out_type=jax.ShapeDtypeStruct((nrows, ltail), jnp.uint32),
        mesh=mesh,
        scratch_types=[
            pltpu.VMEM((_BUF,), jnp.uint32),
            pltpu.SemaphoreType.DMA,
        ],
        compiler_params=pltpu.CompilerParams(use_tc_tiling_on_sc=False),
    )
    return f()


def _body1(x_ref, rl_ref, ov_ref, oi_ref, vb_ref, ib_ref, *, climit, nrows, grid):
    j = pl.program_id(0)
    nsub = _CHUNK // _SUB

    @pl.when(j == 0)
    def _init():
        vb_ref[...] = jnp.full((nrows, _SUB), _NEG_BIG, jnp.float32)
        ib_ref[...] = jnp.zeros((nrows, _SUB), jnp.int32)

    rowlane = rl_ref[...]

    def make_sub(masked):
        def sub(s, carry):
            vb, ib = carry
            base = j * _CHUNK + s * _SUB
            x = x_ref[:, pl.ds(s * _SUB, _SUB)]
            i = rowlane + base.astype(jnp.uint32)
            g = _gumbel_from_bits(_threefry_bits(i))
            v = x + g
            if masked:
                lane = lax.broadcasted_iota(jnp.int32, (nrows, _SUB), 1)
                v = jnp.where(lane + base < climit, v, _NEG_BIG)
            ib = jnp.where(v > vb, base, ib)
            vb = jnp.maximum(vb, v)
            return vb, ib

        return sub

    carry0 = (vb_ref[...], ib_ref[...])
    ragged = climit % _CHUNK != 0

    @pl.when(j < grid - 1)
    def _full():
        vb, ib = lax.fori_loop(0, nsub, make_sub(False), carry0, unroll=16)
        vb_ref[...] = vb
        ib_ref[...] = ib

    @pl.when(j == grid - 1)
    def _last():
        vb, ib = lax.fori_loop(0, nsub, make_sub(ragged), carry0, unroll=16)
        lane = lax.broadcasted_iota(jnp.int32, (nrows, _SUB), 1)
        m = jnp.max(vb, axis=1, keepdims=True)
        cand = jnp.where(vb == m, ib + lane, jnp.int32(climit))
        ov_ref[...] = m
        oi_ref[...] = jnp.min(cand, axis=1, keepdims=True)


def _tc1(outputs, climit):
    nrows, ncols = outputs.shape
    grid = pl.cdiv(climit, _CHUNK)
    rowlane = (
        jnp.arange(nrows, dtype=jnp.uint32)[:, None] * jnp.uint32(ncols)
        + jnp.arange(_SUB, dtype=jnp.uint32)[None, :]
    )
    return pl.pallas_call(
        functools.partial(_body1, climit=climit, nrows=nrows, grid=grid),
        grid=(grid,),
        in_specs=[
            pl.BlockSpec((nrows, _CHUNK), lambda j: (0, j)),
            pl.BlockSpec((nrows, _SUB), lambda j: (0, 0)),
        ],
        out_specs=[
            pl.BlockSpec((nrows, 1), lambda j: (0, 0)),
            pl.BlockSpec((nrows, 1), lambda j: (0, 0)),
        ],
        out_shape=[
            jax.ShapeDtypeStruct((nrows, 1), jnp.float32),
            jax.ShapeDtypeStruct((nrows, 1), jnp.int32),
        ],
        scratch_shapes=[
            pltpu.VMEM((nrows, _SUB), jnp.float32),
            pltpu.VMEM((nrows, _SUB), jnp.int32),
        ],
        compiler_params=pltpu.CompilerParams(
            dimension_semantics=("arbitrary",),
        ),
    )(outputs, rowlane)


def _body2(x_ref, b_ref, v1_ref, i1_ref, o_ref, vb_ref, ib_ref, *, ncols, c0, nrows, grid):
    j = pl.program_id(0)
    nsub = _CHUNK // _SUB

    @pl.when(j == 0)
    def _init():
        vb_ref[...] = jnp.full((nrows, _SUB), _NEG_BIG, jnp.float32)
        ib_ref[...] = jnp.zeros((nrows, _SUB), jnp.int32)

    def make_sub(masked):
        def sub(s, carry):
            vb, ib = carry
            base = c0 + j * _CHUNK + s * _SUB
            x = x_ref[:, pl.ds(s * _SUB, _SUB)]
            bits = b_ref[:, pl.ds(s * _SUB, _SUB)]
            g = _gumbel_from_bits(bits)
            v = x + g
            if masked:
                lane = lax.broadcasted_iota(jnp.int32, (nrows, _SUB), 1)
                v = jnp.where(lane + base < ncols, v, _NEG_BIG)
            ib = jnp.where(v > vb, base, ib)
            vb = jnp.maximum(vb, v)
            return vb, ib

        return sub

    carry0 = (vb_ref[...], ib_ref[...])
    ragged = (ncols - c0) % _CHUNK != 0

    @pl.when(j < grid - 1)
    def _full():
        vb, ib = lax.fori_loop(0, nsub, make_sub(False), carry0, unroll=8)
        vb_ref[...] = vb
        ib_ref[...] = ib

    @pl.when(j == grid - 1)
    def _last():
        vb, ib = lax.fori_loop(0, nsub, make_sub(ragged), carry0, unroll=8)
        lane = lax.broadcasted_iota(jnp.int32, (nrows, _SUB), 1)
        m2 = jnp.max(vb, axis=1, keepdims=True)
        cand = jnp.where(vb == m2, ib + lane, jnp.int32(ncols))
        i2 = jnp.min(cand, axis=1, keepdims=True)
        o_ref[...] = jnp.where(m2 > v1_ref[...], i2, i1_ref[...])


def _tc2(outputs, bits, v1, i1, c0):
    nrows, ncols = outputs.shape
    ltail = ncols - c0
    grid = pl.cdiv(ltail, _CHUNK)
    cb = c0 // _CHUNK
    return pl.pallas_call(
        functools.partial(_body2, ncols=ncols, c0=c0, nrows=nrows, grid=grid),
        grid=(grid,),
        in_specs=[
            pl.BlockSpec((nrows, _CHUNK), lambda j: (0, cb + j)),
            pl.BlockSpec((nrows, _CHUNK), lambda j: (0, j)),
            pl.BlockSpec((nrows, 1), lambda j: (0, 0)),
            pl.BlockSpec((nrows, 1), lambda j: (0, 0)),
        ],
        out_specs=pl.BlockSpec((nrows, 1), lambda j: (0, 0)),
        out_shape=jax.ShapeDtypeStruct((nrows, 1), jnp.int32),
        scratch_shapes=[
            pltpu.VMEM((nrows, _SUB), jnp.float32),
            pltpu.VMEM((nrows, _SUB), jnp.int32),
        ],
        compiler_params=pltpu.CompilerParams(
            dimension_semantics=("arbitrary",),
        ),
    )(outputs, bits, v1, i1)


@jax.jit
def kernel(outputs):
    nrows, ncols = outputs.shape
    if nrows == 32 and ncols > 2 * _CHUNK + _SC_TAIL:
        c0 = ((ncols - _SC_TAIL) // _CHUNK) * _CHUNK
        ltail = ncols - c0
        bits = _sc_bits(nrows, ncols, c0, ltail)
        v1, i1 = _tc1(outputs, c0)
        return _tc2(outputs, bits, v1, i1, c0)
    v1, i1 = _tc1(outputs, ncols)
    return i1


# speedup vs baseline: 2.2135x; 2.2135x over previous
import functools

import jax
import jax.numpy as jnp
from jax import lax
from jax.experimental import pallas as pl
from jax.experimental.pallas import tpu as pltpu
import jax.experimental.pallas.tpu_sc as plsc

_ROT0 = (13, 15, 26, 6)
_ROT1 = (17, 29, 16, 24)
_TINY = float.fromhex("0x1p-126")
_NEG_BIG = -3.0e38
_SUB = 256
_CHUNK = 32768
_BUF = 16384
_SC_TAIL = 279104


def _log2(x):
    return jnp.log2(x)


def _rotl(x, d):
    return lax.shift_left(x, jnp.uint32(d)) | lax.shift_right_logical(
        x, jnp.uint32(32 - d)
    )


def _threefry_bits(i):
    ks = (0, 42, 42 ^ 0x1BD11BDA)
    x1 = i + jnp.uint32(42)
    x0 = x1
    x1 = _rotl(x1, _ROT0[0]) ^ x0
    for r in _ROT0[1:]:
        x0 = x0 + x1
        x1 = _rotl(x1, r)
        x1 = x1 ^ x0
    x0 = x0 + jnp.uint32(ks[1] & 0xFFFFFFFF)
    x1 = x1 + jnp.uint32((ks[2] + 1) & 0xFFFFFFFF)
    for g in range(1, 5):
        rots = _ROT0 if g % 2 == 0 else _ROT1
        for r in rots:
            x0 = x0 + x1
            x1 = _rotl(x1, r)
            x1 = x1 ^ x0
        x0 = x0 + jnp.uint32(ks[(g + 1) % 3] & 0xFFFFFFFF)
        x1 = x1 + jnp.uint32((ks[(g + 2) % 3] + g + 1) & 0xFFFFFFFF)
    return x0 ^ x1


def _gumbel_from_bits(bits):
    fb = lax.shift_right_logical(bits, jnp.uint32(9)) | jnp.uint32(0x3F800000)
    u = lax.bitcast_convert_type(fb, jnp.float32) - jnp.float32(1.0)
    uu = u + _TINY
    nln2 = jnp.float32(-0.6931471805599453)
    m1 = _log2(uu) * nln2
    return _log2(m1) * nln2


def _sc_bits_body(o_ref, buf_ref, sem, *, ncols, c0, ncg_band):
    c = lax.axis_index("c")
    s = lax.axis_index("s")
    sid = c * 16 + s
    rg = sid // 8
    b = sid % 8
    band0 = b * ncg_band
    iota16 = lax.iota(jnp.int32, 16)
    ncg_buf = _BUF // 1024
    nfull = ncg_band // ncg_buf
    rem = ncg_band - nfull * ncg_buf

    def emit(cg_off, n_cg):
        @plsc.parallel_loop(0, n_cg * 64, unroll=4)
        def _gen(w):
            t = w // 64
            rr = (w % 64) // 8
            k = w % 8
            col = c0 + (band0 + cg_off + t) * 128 + k * 16
            ctr = ((rg * 8 + rr) * ncols + col + iota16).astype(jnp.uint32)
            buf_ref[t, rr, pl.ds(k * 16, 16)] = _threefry_bits(ctr)

        cp = pltpu.make_async_copy(
            buf_ref.at[pl.ds(0, n_cg)],
            o_ref.at[rg, pl.ds(band0 + cg_off, n_cg)],
            sem,
        )
        cp.start()
        cp.wait()

    def chunk(k, carry):
        emit(k * ncg_buf, ncg_buf)
        return carry

    lax.fori_loop(0, nfull, chunk, 0)
    if rem:
        emit(nfull * ncg_buf, rem)


def _sc_bits(nrows, ncols, c0, ltail_pad):
    ncg_pad = ltail_pad // 128
    ncg_band = ncg_pad // 8
    mesh = plsc.VectorSubcoreMesh(core_axis_name="c", subcore_axis_name="s")
    f = pl.kernel(
        functools.partial(_sc_bits_body, ncols=ncols, c0=c0, ncg_band=ncg_band),
        out_type=jax.ShapeDtypeStruct((nrows // 8, ncg_pad, 8, 128), jnp.uint32),
        mesh=mesh,
        scratch_types=[
            pltpu.VMEM((_BUF // 1024, 8, 128), jnp.uint32),
            pltpu.SemaphoreType.DMA,
        ],
    )
    return f()


def _body1(x_ref, rl_ref, ov_ref, oi_ref, vb_ref, ib_ref, *, climit, nrows, grid):
    j = pl.program_id(0)
    nsub = _CHUNK // _SUB

    @pl.when(j == 0)
    def _init():
        vb_ref[...] = jnp.full((nrows, _SUB), _NEG_BIG, jnp.float32)
        ib_ref[...] = jnp.zeros((nrows, _SUB), jnp.int32)

    rowlane = rl_ref[...]

    def make_sub(masked):
        def sub(s, carry):
            vb, ib = carry
            base = j * _CHUNK + s * _SUB
            x = x_ref[:, pl.ds(s * _SUB, _SUB)]
            i = rowlane + base.astype(jnp.uint32)
            g = _gumbel_from_bits(_threefry_bits(i))
            v = x + g
            if masked:
                lane = lax.broadcasted_iota(jnp.int32, (nrows, _SUB), 1)
                v = jnp.where(lane + base < climit, v, _NEG_BIG)
            ib = jnp.where(v > vb, base, ib)
            vb = jnp.maximum(vb, v)
            return vb, ib

        return sub

    carry0 = (vb_ref[...], ib_ref[...])
    ragged = climit % _CHUNK != 0

    @pl.when(j < grid - 1)
    def _full():
        vb, ib = lax.fori_loop(0, nsub, make_sub(False), carry0, unroll=16)
        vb_ref[...] = vb
        ib_ref[...] = ib

    @pl.when(j == grid - 1)
    def _last():
        vb, ib = lax.fori_loop(0, nsub, make_sub(ragged), carry0, unroll=16)
        lane = lax.broadcasted_iota(jnp.int32, (nrows, _SUB), 1)
        m = jnp.max(vb, axis=1, keepdims=True)
        cand = jnp.where(vb == m, ib + lane, jnp.int32(climit))
        ov_ref[...] = m
        oi_ref[...] = jnp.min(cand, axis=1, keepdims=True)


def _tc1(outputs, climit):
    nrows, ncols = outputs.shape
    grid = pl.cdiv(climit, _CHUNK)
    rowlane = (
        jnp.arange(nrows, dtype=jnp.uint32)[:, None] * jnp.uint32(ncols)
        + jnp.arange(_SUB, dtype=jnp.uint32)[None, :]
    )
    return pl.pallas_call(
        functools.partial(_body1, climit=climit, nrows=nrows, grid=grid),
        grid=(grid,),
        in_specs=[
            pl.BlockSpec((nrows, _CHUNK), lambda j: (0, j)),
            pl.BlockSpec((nrows, _SUB), lambda j: (0, 0)),
        ],
        out_specs=[
            pl.BlockSpec((nrows, 1), lambda j: (0, 0)),
            pl.BlockSpec((nrows, 1), lambda j: (0, 0)),
        ],
        out_shape=[
            jax.ShapeDtypeStruct((nrows, 1), jnp.float32),
            jax.ShapeDtypeStruct((nrows, 1), jnp.int32),
        ],
        scratch_shapes=[
            pltpu.VMEM((nrows, _SUB), jnp.float32),
            pltpu.VMEM((nrows, _SUB), jnp.int32),
        ],
        compiler_params=pltpu.CompilerParams(
            dimension_semantics=("arbitrary",),
        ),
    )(outputs, rowlane)


def _body2(x_ref, b_ref, v1_ref, i1_ref, o_ref, vb_ref, ib_ref, *, ncols, c0, nrows, grid):
    j = pl.program_id(0)
    nsub = _CHUNK // _SUB

    @pl.when(j == 0)
    def _init():
        vb_ref[...] = jnp.full((nrows, _SUB), _NEG_BIG, jnp.float32)
        ib_ref[...] = jnp.zeros((nrows, _SUB), jnp.int32)

    def make_sub(masked):
        def sub(s, carry):
            vb, ib = carry
            base = c0 + j * _CHUNK + s * _SUB
            x = x_ref[:, pl.ds(s * _SUB, _SUB)]
            bits = b_ref[:, pl.ds(s * _SUB, _SUB)]
            g = _gumbel_from_bits(bits)
            v = x + g
            if masked:
                lane = lax.broadcasted_iota(jnp.int32, (nrows, _SUB), 1)
                v = jnp.where(lane + base < ncols, v, _NEG_BIG)
            ib = jnp.where(v > vb, base, ib)
            vb = jnp.maximum(vb, v)
            return vb, ib

        return sub

    carry0 = (vb_ref[...], ib_ref[...])
    ragged = (ncols - c0) % _CHUNK != 0

    @pl.when(j < grid - 1)
    def _full():
        vb, ib = lax.fori_loop(0, nsub, make_sub(False), carry0, unroll=8)
        vb_ref[...] = vb
        ib_ref[...] = ib

    @pl.when(j == grid - 1)
    def _last():
        vb, ib = lax.fori_loop(0, nsub, make_sub(ragged), carry0, unroll=8)
        lane = lax.broadcasted_iota(jnp.int32, (nrows, _SUB), 1)
        m2 = jnp.max(vb, axis=1, keepdims=True)
        cand = jnp.where(vb == m2, ib + lane, jnp.int32(ncols))
        i2 = jnp.min(cand, axis=1, keepdims=True)
        o_ref[...] = jnp.where(m2 > v1_ref[...], i2, i1_ref[...])


def _tc2(outputs, bits, v1, i1, c0):
    nrows, ncols = outputs.shape
    ltail = ncols - c0
    grid = pl.cdiv(ltail, _CHUNK)
    cb = c0 // _CHUNK
    return pl.pallas_call(
        functools.partial(_body2, ncols=ncols, c0=c0, nrows=nrows, grid=grid),
        grid=(grid,),
        in_specs=[
            pl.BlockSpec((nrows, _CHUNK), lambda j: (0, cb + j)),
            pl.BlockSpec((nrows, _CHUNK), lambda j: (0, j)),
            pl.BlockSpec((nrows, 1), lambda j: (0, 0)),
            pl.BlockSpec((nrows, 1), lambda j: (0, 0)),
        ],
        out_specs=pl.BlockSpec((nrows, 1), lambda j: (0, 0)),
        out_shape=jax.ShapeDtypeStruct((nrows, 1), jnp.int32),
        scratch_shapes=[
            pltpu.VMEM((nrows, _SUB), jnp.float32),
            pltpu.VMEM((nrows, _SUB), jnp.int32),
        ],
        compiler_params=pltpu.CompilerParams(
            dimension_semantics=("arbitrary",),
        ),
    )(outputs, bits, v1, i1)


@jax.jit
def kernel(outputs):
    nrows, ncols = outputs.shape
    if nrows == 32 and ncols > 2 * _CHUNK + _SC_TAIL:
        c0 = ((ncols - _SC_TAIL) // _CHUNK) * _CHUNK
        ltail = ncols - c0
        ltail_pad = ((ltail + 1023) // 1024) * 1024
        bits4 = _sc_bits(nrows, ncols, c0, ltail_pad)
        v1, i1 = _tc1(outputs, c0)
        bits = bits4.transpose(0, 2, 1, 3).reshape(nrows, ltail_pad)
        return _tc2(outputs, bits, v1, i1, c0)
    v1, i1 = _tc1(outputs, ncols)
    return i1
